# baseline (speedup 1.0000x reference)
import functools

import jax
import jax.numpy as jnp
from jax import lax
from jax.experimental import pallas as pl
from jax.experimental.pallas import tpu as pltpu

_RESIDENT_MAX = 2048

_K = 4

_DIMS = (((1,), (1,)), ((), ()))


def _dott(lhs, rhs):
    return lax.dot_general(lhs, rhs, _DIMS, preferred_element_type=jnp.float32)


def _prop_body(a2_ref, a1_ref, ht_ref, out_ref, buf, sem, a2_sc, a1_sc,
               t1, e1, t2, *, bm, grid_rows, resident_a1):
    p = pl.program_id(0)
    i = pl.program_id(1)
    rows = pl.ds(i * bm, bm)
    g = grid_rows

    hm = bm // 2

    def _half_copies(src_ref, strip, slot):
        return [
            pltpu.make_async_copy(
                src_ref.at[pl.ds(strip * bm + h * hm, hm), :],
                buf.at[slot, pl.ds(h * hm, hm), :],
                sem.at[slot, h],
            )
            for h in (0, 1)
        ]

    def issue(pos, slot):
        @pl.when(pos < g)
        def _from_a2():
            for c in _half_copies(a2_ref, pos, slot):
                c.start()

        @pl.when(pos >= g)
        def _from_a1():
            strip = jnp.where(pos < 2 * g, pos - g, pos - 2 * g)
            for c in _half_copies(a1_ref, strip, slot):
                c.start()

    n_pos = 2 * g if resident_a1 else 3 * g
    consuming = (p == 0) | (p == 1) | ((p == 3) & (not resident_a1))
    pos = jnp.where(p == 0, i, jnp.where(p == 1, g + i, 2 * g + i))
    slot = lax.rem(pos, _K)

    @pl.when((p == 0) & (i == 0))
    def _prologue():
        for k in range(_K):
            issue(jnp.int32(k), jnp.int32(k))

    def wait_and_issue_next():
        for c in _half_copies(a2_ref, jnp.int32(0), slot):
            c.wait()

    def refill():
        nxt = pos + _K

        @pl.when(consuming & (nxt < n_pos))
        def _():
            issue(nxt, slot)

    @pl.when(p == 0)
    def _phase0():
        wait_and_issue_next()
        blk = buf[slot].astype(jnp.bfloat16)
        a2_sc[rows, :] = blk
        t1[:, rows] = _dott(ht_ref[...].astype(jnp.bfloat16), blk)

    @pl.when(p == 1)
    def _phase1():
        wait_and_issue_next()
        blk = buf[slot].astype(jnp.bfloat16)
        if resident_a1:
            a1_sc[rows, :] = blk
        e1[:, rows] = _dott(t1[...].astype(jnp.bfloat16), blk)

    @pl.when(p == 2)
    def _phase2():
        t2[:, rows] = _dott(e1[...].astype(jnp.bfloat16), a2_sc[rows, :])

    @pl.when(p == 3)
    def _phase3():
        if resident_a1:
            blk = a1_sc[rows, :]
        else:
            wait_and_issue_next()
            blk = buf[slot].astype(jnp.bfloat16)
        e2_blk = _dott(t2[...].astype(jnp.bfloat16), blk)
        out_ref[...] = (ht_ref[:, rows] + e1[:, rows] + e2_blk) * (1.0 / 3.0)

    refill()


def _prop(a1, a2, h, *, bm):
    n, d = h.shape
    grid_rows = n // bm
    resident_a1 = n <= _RESIDENT_MAX

    def out_map(p, i):
        return (0, jnp.where(p == 3, i, 0))

    scratch = [
        pltpu.VMEM((_K, bm, n), jnp.float32),
        pltpu.SemaphoreType.DMA((_K, 2)),
        pltpu.VMEM((n, n), jnp.bfloat16),
        pltpu.VMEM((n, n) if resident_a1 else (8, 128), jnp.bfloat16),
        pltpu.VMEM((d, n), jnp.float32),
        pltpu.VMEM((d, n), jnp.float32),
        pltpu.VMEM((d, n), jnp.float32),
    ]

    ht = h.T
    outt = pl.pallas_call(
        functools.partial(_prop_body, bm=bm, grid_rows=grid_rows,
                          resident_a1=resident_a1),
        grid=(4, grid_rows),
        in_specs=[
            pl.BlockSpec(memory_space=pl.ANY),
            pl.BlockSpec(memory_space=pl.ANY),
            pl.BlockSpec((d, n), lambda p, i: (0, 0)),
        ],
        out_specs=pl.BlockSpec((d, bm), out_map),
        out_shape=jax.ShapeDtypeStruct((d, n), jnp.float32),
        scratch_shapes=scratch,
        compiler_params=pltpu.CompilerParams(
            dimension_semantics=("arbitrary", "arbitrary"),
        ),
    )(a2, a1, ht)
    return outt.T


def kernel(adj_u1, adj_u2, adj_i1, adj_i2, adj_m1, adj_m2, adj_a1, adj_a2,
           user_emb, item_emb, mashup_tag_emb, api_tag_emb):
    u = _prop(adj_u1, adj_u2, user_emb, bm=256)
    i = _prop(adj_i1, adj_i2, item_emb, bm=256)
    m = _prop(adj_m1, adj_m2, mashup_tag_emb, bm=256)
    a = _prop(adj_a1, adj_a2, api_tag_emb, bm=256)
    return (u, i, m, a)
